# baseline (speedup 1.0000x reference)
import functools

import jax
from jax import lax
import jax.numpy as jnp
import numpy as np
from jax.experimental import pallas as pl
from jax.experimental.pallas import tpu as pltpu

_SCORE_THRESH = 0.05
_NMS_THRESH = 0.5
_DETS = 100
_SCALE = 1.0 / 16.0
_POOL = 7
_NCLS = 91
_LANES = 128

_NPROP = 5000
_NPTS = 5120 * 49
_TILES = 32
_TPTS = _NPTS // _TILES
_CH = 160
_NSUB = _TPTS // _CH
_NPROPP = 5120
_BLK = 512
_ROWS = _NPROPP


def _sc_roi_body(table_hbm, px1_hbm, py1_hbm, px2_hbm, py2_hbm, out_hbm,
                 shared_v, px1_v, py1_v, px2_v, py2_v, i00_v, i01_v, i10_v,
                 i11_v, w_v, c00_v, c01_v, c10_v, c11_v, out_v, sem):
    from jax.experimental.pallas import tpu_sc as plsc
    wid = lax.axis_index("s") * 2 + lax.axis_index("c")

    @pl.when(lax.axis_index("s") == 0)
    def _stage():
        pltpu.sync_copy(table_hbm, shared_v)

    pltpu.sync_copy(px1_hbm, px1_v)
    pltpu.sync_copy(py1_hbm, py1_v)
    pltpu.sync_copy(px2_hbm, px2_v)
    pltpu.sync_copy(py2_hbm, py2_v)
    base = wid * _TPTS
    iota16 = lax.iota(jnp.int32, 16)
    plsc.subcore_barrier()

    def subchunk(scix, carry):
        cb = base + scix * _CH

        def group(g, carry2):
            pt = cb + g * 16 + iota16
            n = lax.div(pt, 49)
            p = pt - n * 49
            py = lax.div(p, 7)
            px = p - py * 7
            x1 = plsc.load_gather(px1_v, [n]) * _SCALE
            y1 = plsc.load_gather(py1_v, [n]) * _SCALE
            x2 = plsc.load_gather(px2_v, [n]) * _SCALE
            y2 = plsc.load_gather(py2_v, [n]) * _SCALE
            bw = jnp.maximum(x2 - x1, 1.0)
            bh = jnp.maximum(y2 - y1, 1.0)
            gx = (x1 + (px.astype(jnp.float32) + 0.5) * (bw / 7.0)) - 0.5
            gy = (y1 + (py.astype(jnp.float32) + 0.5) * (bh / 7.0)) - 0.5

            def fl(v):
                t = v.astype(jnp.int32)
                tf = t.astype(jnp.float32)
                return jnp.where((v < 0.0) & (tf != v), t - 1, t)

            x0i = fl(gx)
            y0i = fl(gy)
            lx = jnp.clip(gx - x0i.astype(jnp.float32), 0.0, 1.0)
            ly = jnp.clip(gy - y0i.astype(jnp.float32), 0.0, 1.0)
            x0c = jnp.clip(x0i, 0, 49)
            x1c = jnp.clip(x0i + 1, 0, 49)
            y0c = jnp.clip(y0i, 0, 49)
            y1c = jnp.clip(y0i + 1, 0, 49)
            sl = pl.ds(g * 16, 16)
            i00_v[sl] = y0c * 50 + x0c
            i01_v[sl] = y0c * 50 + x1c
            i10_v[sl] = y1c * 50 + x0c
            i11_v[sl] = y1c * 50 + x1c
            w_v[0, sl] = (1.0 - ly) * (1.0 - lx)
            w_v[1, sl] = (1.0 - ly) * lx
            w_v[2, sl] = ly * (1.0 - lx)
            w_v[3, sl] = ly * lx
            return carry2

        @plsc.parallel_loop(0, _CH // 16, unroll=2)
        def _g(g):
            group(g, 0)

        h00 = pltpu.async_copy(shared_v.at[i00_v], c00_v, sem)
        h01 = pltpu.async_copy(shared_v.at[i01_v], c01_v, sem)
        h10 = pltpu.async_copy(shared_v.at[i10_v], c10_v, sem)
        h11 = pltpu.async_copy(shared_v.at[i11_v], c11_v, sem)
        h00.wait()
        h01.wait()
        h10.wait()
        h11.wait()

        def pgroup(g, carry3):
            sl = pl.ds(g * 16, 16)
            wv00 = w_v[0, sl]
            wv01 = w_v[1, sl]
            wv10 = w_v[2, sl]
            wv11 = w_v[3, sl]
            for j in range(16):
                i = g * 16 + j
                for v in range(4):
                    cs = pl.ds(v * 16, 16)
                    out_v[i, cs] = (c00_v[i, cs] * wv00[j] + c01_v[i, cs] * wv01[j]
                                    + c10_v[i, cs] * wv10[j] + c11_v[i, cs] * wv11[j])
            return carry3

        @plsc.parallel_loop(0, _CH // 16, unroll=4)
        def _pg(g):
            pgroup(g, 0)

        pltpu.sync_copy(out_v, out_hbm.at[pl.ds(cb, _CH)])
        return carry

    lax.fori_loop(0, _NSUB, subchunk, 0)


@jax.jit
def _sc_roi(table, prop):
    from jax.experimental.pallas import tpu_sc as plsc
    f32 = jnp.float32
    fn = functools.partial(
        pl.kernel,
        mesh=plsc.VectorSubcoreMesh(core_axis_name="c", subcore_axis_name="s"),
        compiler_params=pltpu.CompilerParams(
            needs_layout_passes=False, use_tc_tiling_on_sc=False),
        out_type=jax.ShapeDtypeStruct((_NPTS, 64), f32),
        scratch_types=[
            pltpu.VMEM_SHARED((2500, 64), f32),
            pltpu.VMEM((_NPROPP,), f32),
            pltpu.VMEM((_NPROPP,), f32),
            pltpu.VMEM((_NPROPP,), f32),
            pltpu.VMEM((_NPROPP,), f32),
            pltpu.VMEM((_CH,), jnp.int32),
            pltpu.VMEM((_CH,), jnp.int32),
            pltpu.VMEM((_CH,), jnp.int32),
            pltpu.VMEM((_CH,), jnp.int32),
            pltpu.VMEM((4, _CH), f32),
            pltpu.VMEM((_CH, 64), f32),
            pltpu.VMEM((_CH, 64), f32),
            pltpu.VMEM((_CH, 64), f32),
            pltpu.VMEM((_CH, 64), f32),
            pltpu.VMEM((_CH, 64), f32),
            pltpu.SemaphoreType.DMA,
        ],
    )(_sc_roi_body)
    return fn(table, prop[:, 0], prop[:, 1], prop[:, 2], prop[:, 3])


def _head_body(x_ref, prop_ref, w6_ref, b6_ref, w7_ref, b7_ref,
               wc_ref, bc_ref, wr_ref, br_ref, bounds_ref,
               bx_ref, ms_ref):
    pid = pl.program_id(0)
    x = x_ref[...]
    a = jnp.maximum(
        jnp.dot(x, w6_ref[...], preferred_element_type=jnp.float32) + b6_ref[...], 0.0)
    h = jnp.maximum(
        jnp.dot(a, w7_ref[...], preferred_element_type=jnp.float32) + b7_ref[...], 0.0)
    lg = jnp.dot(h, wc_ref[...], preferred_element_type=jnp.float32) + bc_ref[...]
    lane = jax.lax.broadcasted_iota(jnp.int32, (_BLK, _LANES), 1)
    lg = jnp.where(lane < _NCLS, lg, -jnp.inf)
    e = jnp.exp(lg - jnp.max(lg, axis=1, keepdims=True))
    sm = e / jnp.sum(e, axis=1, keepdims=True)
    sc = sm[:, 1:_NCLS]
    rel = jnp.dot(h, wr_ref[...], preferred_element_type=jnp.float32) + br_ref[...]

    px1 = prop_ref[:, 0:1]
    py1 = prop_ref[:, 1:2]
    px2 = prop_ref[:, 2:3]
    py2 = prop_ref[:, 3:4]
    widths = px2 - px1
    heights = py2 - py1
    ctr_x = px1 + 0.5 * widths
    ctr_y = py1 + 0.5 * heights
    dx = rel[:, 0:90] / 10.0
    dy = rel[:, 96:186] / 10.0
    dw = jnp.minimum(rel[:, 192:282] / 5.0, np.log(1000.0 / 16))
    dh = jnp.minimum(rel[:, 288:378] / 5.0, np.log(1000.0 / 16))
    pcx = dx * widths + ctr_x
    pcy = dy * heights + ctr_y
    pw = jnp.exp(dw) * widths
    ph = jnp.exp(dh) * heights
    img_w = bounds_ref[0, 0]
    img_h = bounds_ref[0, 1]
    bx1 = jnp.clip(pcx - 0.5 * pw, 0.0, img_w)
    by1 = jnp.clip(pcy - 0.5 * ph, 0.0, img_h)
    bx2 = jnp.clip(pcx + 0.5 * pw, 0.0, img_w)
    by2 = jnp.clip(pcy + 0.5 * ph, 0.0, img_h)

    ws = bx2 - bx1
    hs = by2 - by1
    valid = (sc > _SCORE_THRESH) & (ws >= 0.01) & (hs >= 0.01)
    msk = jnp.where(valid, sc, -1.0)
    row = pid * _BLK + jax.lax.broadcasted_iota(jnp.int32, (_BLK, _NCLS - 1), 0)
    real = row < _NPROP
    msk = jnp.where(real, msk, -1.0)
    zero = jnp.zeros_like(bx1)
    lpad = jnp.zeros((_BLK, _LANES - (_NCLS - 1)), jnp.float32)
    def padl(p):
        return jnp.concatenate([p, lpad], axis=1)
    bx_ref[0] = padl(jnp.where(real, bx1, zero))
    bx_ref[1] = padl(jnp.where(real, by1, zero))
    bx_ref[2] = padl(jnp.where(real, bx2, zero))
    bx_ref[3] = padl(jnp.where(real, by2, zero))
    ms_ref[...] = padl(msk)


@jax.jit
def _head_pallas(x, prop, w6, b6, w7, b7, wc, bc, wr, br, bounds):
    f32 = jnp.float32
    nblk = _NPROPP // _BLK
    full = lambda shape: pl.BlockSpec(shape, lambda i: (0,) * len(shape))
    return pl.pallas_call(
        _head_body,
        grid=(nblk,),
        in_specs=[
            pl.BlockSpec((_BLK, 3136), lambda i: (i, 0)),
            pl.BlockSpec((_BLK, 4), lambda i: (i, 0)),
            full((3136, 256)),
            full((1, 256)),
            full((256, 256)),
            full((1, 256)),
            full((256, _LANES)),
            full((1, _LANES)),
            full((256, 384)),
            full((1, 384)),
            full((1, _LANES)),
        ],
        out_specs=[
            pl.BlockSpec((4, _BLK, _LANES), lambda i: (0, i, 0)),
            pl.BlockSpec((_BLK, _LANES), lambda i: (i, 0)),
        ],
        out_shape=[
            jax.ShapeDtypeStruct((4, _ROWS, _LANES), f32),
            jax.ShapeDtypeStruct((_ROWS, _LANES), f32),
        ],
    )(x, prop, w6, b6, w7, b7, wc, bc, wr, br, bounds)


def _nms_body(bxT_ref, msT_ref, outb_ref, outs_ref, outl_ref,
              nbT_ref, sT_ref, a2T_ref, rmax_ref, rix_ref):
    big = jnp.int32(2**30)
    m = jnp.max(bxT_ref[...])
    c_col = jax.lax.broadcasted_iota(jnp.int32, (_LANES, 1), 0)
    offsT = (c_col + 1).astype(jnp.float32) * (m + 1.0)
    for k in range(4):
        nbT_ref[k] = bxT_ref[k] + offsT
    a2T_ref[...] = (nbT_ref[2] - nbT_ref[0]) * (nbT_ref[3] - nbT_ref[1])
    sT_ref[...] = msT_ref[...]
    piota = jax.lax.broadcasted_iota(jnp.int32, (_LANES, _ROWS), 1)
    rmax_ref[...] = jnp.max(sT_ref[...], axis=1, keepdims=True)
    rix_ref[...] = jnp.min(
        jnp.where(sT_ref[...] == rmax_ref[...], piota, big), axis=1,
        keepdims=True)
    prow = jax.lax.broadcasted_iota(jnp.int32, (1, _ROWS), 1)

    def body(i, carry):
        rmax = rmax_ref[...]
        mv = jnp.max(rmax)
        cand = jnp.where(rmax == mv, rix_ref[...] * 90 + c_col, big)
        idx = jnp.min(cand)
        ok = mv > 0.0
        n = idx // 90
        c = idx - n * 90
        sel = prow == n

        def ext(row):
            return jnp.sum(jnp.where(sel, row, 0.0))

        nrow = [nbT_ref[k, pl.ds(c, 1), :] for k in range(4)]
        px = [ext(r) for r in nrow]
        rx = [ext(bxT_ref[k, pl.ds(c, 1), :]) for k in range(4)]
        area1 = (px[2] - px[0]) * (px[3] - px[1])
        ltx = jnp.maximum(px[0], nrow[0])
        lty = jnp.maximum(px[1], nrow[1])
        rbx = jnp.minimum(px[2], nrow[2])
        rby = jnp.minimum(px[3], nrow[3])
        w = jnp.clip(rbx - ltx, 0.0, None)
        h = jnp.clip(rby - lty, 0.0, None)
        inter = w * h
        iou = inter / (area1 + a2T_ref[pl.ds(c, 1), :] - inter + 1e-9)
        kill = (ok & (iou > _NMS_THRESH)) | sel
        newrow = jnp.where(kill, -1.0, sT_ref[pl.ds(c, 1), :])
        sT_ref[pl.ds(c, 1), :] = newrow
        newmax = jnp.max(newrow)
        newrix = jnp.min(jnp.where(newrow == newmax, prow, big))
        rmax_ref[pl.ds(c, 1), :] = jnp.full((1, 1), 0.0) + newmax
        rix_ref[pl.ds(c, 1), :] = jnp.full((1, 1), 0) + newrix
        for k in range(4):
            outb_ref[k, pl.ds(i, 1), :] = jnp.broadcast_to(
                jnp.where(ok, rx[k], 0.0), (1, _LANES))
        outs_ref[pl.ds(i, 1), :] = jnp.broadcast_to(
            jnp.where(ok, mv, 0.0), (1, _LANES))
        lab = jnp.where(ok, c + 1, 0)
        outl_ref[pl.ds(i, 1), :] = jnp.broadcast_to(lab, (1, _LANES))
        return carry

    jax.lax.fori_loop(0, _DETS, body, 0)


@jax.jit
def _nms_pallas(bx, ms):
    f32 = jnp.float32
    bxT = jnp.transpose(bx, (0, 2, 1))
    msT = ms.T
    return pl.pallas_call(
        _nms_body,
        out_shape=[
            jax.ShapeDtypeStruct((4, _DETS, _LANES), f32),
            jax.ShapeDtypeStruct((_DETS, _LANES), f32),
            jax.ShapeDtypeStruct((_DETS, _LANES), jnp.int32),
        ],
        scratch_shapes=[
            pltpu.VMEM((4, _LANES, _ROWS), f32),
            pltpu.VMEM((_LANES, _ROWS), f32),
            pltpu.VMEM((_LANES, _ROWS), f32),
            pltpu.VMEM((_LANES, 1), f32),
            pltpu.VMEM((_LANES, 1), jnp.int32),
        ],
    )(bxT, msT)


def kernel(features, proposals, fc6_w, fc6_b, fc7_w, fc7_b, cls_w, cls_b,
           reg_w, reg_b, img_h, img_w):
    f32 = jnp.float32
    feat = features[0]
    prop = jnp.concatenate(
        [proposals, jnp.zeros((_NPROPP - _NPROP, 4), f32)], axis=0)
    table = feat.transpose(1, 2, 0).reshape(2500, 64)
    pooled = _sc_roi(table, prop)
    x = pooled.reshape(_NPROPP, 3136)
    b6 = fc6_b.reshape(1, 256)
    b7 = fc7_b.reshape(1, 256)
    wc = jnp.concatenate([cls_w, jnp.zeros((256, _LANES - _NCLS), f32)], axis=1)
    bc = jnp.concatenate([cls_b, jnp.zeros((_LANES - _NCLS,), f32)]).reshape(1, _LANES)
    wr4 = reg_w.reshape(256, _NCLS, 4).transpose(0, 2, 1)[:, :, 1:]
    wr = jnp.concatenate([wr4, jnp.zeros((256, 4, 6), f32)], axis=2).reshape(256, 384)
    br4 = reg_b.reshape(_NCLS, 4).T[:, 1:]
    br = jnp.concatenate([br4, jnp.zeros((4, 6), f32)], axis=1).reshape(1, 384)
    bounds = jnp.zeros((1, _LANES), f32)
    bounds = bounds.at[0, 0].set(jnp.asarray(img_w).astype(f32))
    bounds = bounds.at[0, 1].set(jnp.asarray(img_h).astype(f32))
    w6p = fc6_w.reshape(64, 49, 256).transpose(1, 0, 2).reshape(3136, 256)
    bx, ms = _head_pallas(x, prop, w6p, b6, fc7_w, b7, wc, bc, wr, br, bounds)
    outb, outs, outl = _nms_pallas(bx, ms)
    out_boxes = outb[:, :, 0].T
    out_scores = outs[:, 0]
    out_labels = outl[:, 0]
    return out_boxes, out_scores, out_labels


# speedup vs baseline: 1.4060x; 1.4060x over previous
import functools

import jax
from jax import lax
import jax.numpy as jnp
import numpy as np
from jax.experimental import pallas as pl
from jax.experimental.pallas import tpu as pltpu

_SCORE_THRESH = 0.05
_NMS_THRESH = 0.5
_DETS = 100
_SCALE = 1.0 / 16.0
_POOL = 7
_NCLS = 91
_LANES = 128

_NPROP = 5000
_NPTS = 5120 * 49
_TILES = 32
_TPTS = _NPTS // _TILES
_CH = 160
_NSUB = _TPTS // _CH
_NPROPP = 5120
_BLK = 512
_ROWS = _NPROPP


def _sc_roi_body(table_hbm, px1_hbm, py1_hbm, px2_hbm, py2_hbm, out_hbm,
                 shared_v, px1_v, py1_v, px2_v, py2_v, i00_v, i01_v, i10_v,
                 i11_v, w_v, c00_v, c01_v, c10_v, c11_v, out_v, sem):
    from jax.experimental.pallas import tpu_sc as plsc
    wid = lax.axis_index("s") * 2 + lax.axis_index("c")

    @pl.when(lax.axis_index("s") == 0)
    def _stage():
        pltpu.sync_copy(table_hbm, shared_v)

    pltpu.sync_copy(px1_hbm, px1_v)
    pltpu.sync_copy(py1_hbm, py1_v)
    pltpu.sync_copy(px2_hbm, px2_v)
    pltpu.sync_copy(py2_hbm, py2_v)
    base = wid * _TPTS
    iota16 = lax.iota(jnp.int32, 16)
    plsc.subcore_barrier()

    def subchunk(scix, carry):
        cb = base + scix * _CH

        def group(g, carry2):
            pt = cb + g * 16 + iota16
            n = lax.div(pt, 49)
            p = pt - n * 49
            py = lax.div(p, 7)
            px = p - py * 7
            x1 = plsc.load_gather(px1_v, [n]) * _SCALE
            y1 = plsc.load_gather(py1_v, [n]) * _SCALE
            x2 = plsc.load_gather(px2_v, [n]) * _SCALE
            y2 = plsc.load_gather(py2_v, [n]) * _SCALE
            bw = jnp.maximum(x2 - x1, 1.0)
            bh = jnp.maximum(y2 - y1, 1.0)
            gx = (x1 + (px.astype(jnp.float32) + 0.5) * (bw / 7.0)) - 0.5
            gy = (y1 + (py.astype(jnp.float32) + 0.5) * (bh / 7.0)) - 0.5

            def fl(v):
                t = v.astype(jnp.int32)
                tf = t.astype(jnp.float32)
                return jnp.where((v < 0.0) & (tf != v), t - 1, t)

            x0i = fl(gx)
            y0i = fl(gy)
            lx = jnp.clip(gx - x0i.astype(jnp.float32), 0.0, 1.0)
            ly = jnp.clip(gy - y0i.astype(jnp.float32), 0.0, 1.0)
            x0c = jnp.clip(x0i, 0, 49)
            x1c = jnp.clip(x0i + 1, 0, 49)
            y0c = jnp.clip(y0i, 0, 49)
            y1c = jnp.clip(y0i + 1, 0, 49)
            sl = pl.ds(g * 16, 16)
            i00_v[sl] = y0c * 50 + x0c
            i01_v[sl] = y0c * 50 + x1c
            i10_v[sl] = y1c * 50 + x0c
            i11_v[sl] = y1c * 50 + x1c
            w_v[0, sl] = (1.0 - ly) * (1.0 - lx)
            w_v[1, sl] = (1.0 - ly) * lx
            w_v[2, sl] = ly * (1.0 - lx)
            w_v[3, sl] = ly * lx
            return carry2

        @plsc.parallel_loop(0, _CH // 16, unroll=2)
        def _g(g):
            group(g, 0)

        h00 = pltpu.async_copy(shared_v.at[i00_v], c00_v, sem)
        h01 = pltpu.async_copy(shared_v.at[i01_v], c01_v, sem)
        h10 = pltpu.async_copy(shared_v.at[i10_v], c10_v, sem)
        h11 = pltpu.async_copy(shared_v.at[i11_v], c11_v, sem)
        h00.wait()
        h01.wait()
        h10.wait()
        h11.wait()

        def pgroup(g, carry3):
            sl = pl.ds(g * 16, 16)
            wv00 = w_v[0, sl]
            wv01 = w_v[1, sl]
            wv10 = w_v[2, sl]
            wv11 = w_v[3, sl]
            for j in range(16):
                i = g * 16 + j
                for v in range(4):
                    cs = pl.ds(v * 16, 16)
                    out_v[i, cs] = (c00_v[i, cs] * wv00[j] + c01_v[i, cs] * wv01[j]
                                    + c10_v[i, cs] * wv10[j] + c11_v[i, cs] * wv11[j])
            return carry3

        @plsc.parallel_loop(0, _CH // 16, unroll=2)
        def _pg(g):
            pgroup(g, 0)

        pltpu.sync_copy(out_v, out_hbm.at[pl.ds(cb, _CH)])
        return carry

    lax.fori_loop(0, _NSUB, subchunk, 0)


@jax.jit
def _sc_roi(table, prop):
    from jax.experimental.pallas import tpu_sc as plsc
    f32 = jnp.float32
    fn = functools.partial(
        pl.kernel,
        mesh=plsc.VectorSubcoreMesh(core_axis_name="c", subcore_axis_name="s"),
        compiler_params=pltpu.CompilerParams(
            needs_layout_passes=False, use_tc_tiling_on_sc=False),
        out_type=jax.ShapeDtypeStruct((_NPTS, 64), f32),
        scratch_types=[
            pltpu.VMEM_SHARED((2500, 64), f32),
            pltpu.VMEM((_NPROPP,), f32),
            pltpu.VMEM((_NPROPP,), f32),
            pltpu.VMEM((_NPROPP,), f32),
            pltpu.VMEM((_NPROPP,), f32),
            pltpu.VMEM((_CH,), jnp.int32),
            pltpu.VMEM((_CH,), jnp.int32),
            pltpu.VMEM((_CH,), jnp.int32),
            pltpu.VMEM((_CH,), jnp.int32),
            pltpu.VMEM((4, _CH), f32),
            pltpu.VMEM((_CH, 64), f32),
            pltpu.VMEM((_CH, 64), f32),
            pltpu.VMEM((_CH, 64), f32),
            pltpu.VMEM((_CH, 64), f32),
            pltpu.VMEM((_CH, 64), f32),
            pltpu.SemaphoreType.DMA,
        ],
    )(_sc_roi_body)
    return fn(table, prop[:, 0], prop[:, 1], prop[:, 2], prop[:, 3])


def _head_body(x_ref, prop_ref, w6_ref, b6_ref, w7_ref, b7_ref,
               wc_ref, bc_ref, wr_ref, br_ref, bounds_ref,
               bx_ref, ms_ref):
    pid = pl.program_id(0)
    x = x_ref[...]
    a = jnp.maximum(
        jnp.dot(x, w6_ref[...], preferred_element_type=jnp.float32) + b6_ref[...], 0.0)
    h = jnp.maximum(
        jnp.dot(a, w7_ref[...], preferred_element_type=jnp.float32) + b7_ref[...], 0.0)
    lg = jnp.dot(h, wc_ref[...], preferred_element_type=jnp.float32) + bc_ref[...]
    lane = jax.lax.broadcasted_iota(jnp.int32, (_BLK, _LANES), 1)
    lg = jnp.where(lane < _NCLS, lg, -jnp.inf)
    e = jnp.exp(lg - jnp.max(lg, axis=1, keepdims=True))
    sm = e / jnp.sum(e, axis=1, keepdims=True)
    sc = sm[:, 1:_NCLS]
    rel = jnp.dot(h, wr_ref[...], preferred_element_type=jnp.float32) + br_ref[...]

    px1 = prop_ref[:, 0:1]
    py1 = prop_ref[:, 1:2]
    px2 = prop_ref[:, 2:3]
    py2 = prop_ref[:, 3:4]
    widths = px2 - px1
    heights = py2 - py1
    ctr_x = px1 + 0.5 * widths
    ctr_y = py1 + 0.5 * heights
    dx = rel[:, 0:90] / 10.0
    dy = rel[:, 96:186] / 10.0
    dw = jnp.minimum(rel[:, 192:282] / 5.0, np.log(1000.0 / 16))
    dh = jnp.minimum(rel[:, 288:378] / 5.0, np.log(1000.0 / 16))
    pcx = dx * widths + ctr_x
    pcy = dy * heights + ctr_y
    pw = jnp.exp(dw) * widths
    ph = jnp.exp(dh) * heights
    img_w = bounds_ref[0, 0]
    img_h = bounds_ref[0, 1]
    bx1 = jnp.clip(pcx - 0.5 * pw, 0.0, img_w)
    by1 = jnp.clip(pcy - 0.5 * ph, 0.0, img_h)
    bx2 = jnp.clip(pcx + 0.5 * pw, 0.0, img_w)
    by2 = jnp.clip(pcy + 0.5 * ph, 0.0, img_h)

    ws = bx2 - bx1
    hs = by2 - by1
    valid = (sc > _SCORE_THRESH) & (ws >= 0.01) & (hs >= 0.01)
    msk = jnp.where(valid, sc, -1.0)
    row = pid * _BLK + jax.lax.broadcasted_iota(jnp.int32, (_BLK, _NCLS - 1), 0)
    real = row < _NPROP
    msk = jnp.where(real, msk, -1.0)
    zero = jnp.zeros_like(bx1)
    lpad = jnp.zeros((_BLK, _LANES - (_NCLS - 1)), jnp.float32)
    def padl(p):
        return jnp.concatenate([p, lpad], axis=1)
    bx_ref[0] = padl(jnp.where(real, bx1, zero))
    bx_ref[1] = padl(jnp.where(real, by1, zero))
    bx_ref[2] = padl(jnp.where(real, bx2, zero))
    bx_ref[3] = padl(jnp.where(real, by2, zero))
    ms_ref[...] = padl(msk)


@jax.jit
def _head_pallas(x, prop, w6, b6, w7, b7, wc, bc, wr, br, bounds):
    f32 = jnp.float32
    nblk = _NPROPP // _BLK
    full = lambda shape: pl.BlockSpec(shape, lambda i: (0,) * len(shape))
    return pl.pallas_call(
        _head_body,
        grid=(nblk,),
        in_specs=[
            pl.BlockSpec((_BLK, 3136), lambda i: (i, 0)),
            pl.BlockSpec((_BLK, 4), lambda i: (i, 0)),
            full((3136, 256)),
            full((1, 256)),
            full((256, 256)),
            full((1, 256)),
            full((256, _LANES)),
            full((1, _LANES)),
            full((256, 384)),
            full((1, 384)),
            full((1, _LANES)),
        ],
        out_specs=[
            pl.BlockSpec((4, _BLK, _LANES), lambda i: (0, i, 0)),
            pl.BlockSpec((_BLK, _LANES), lambda i: (i, 0)),
        ],
        out_shape=[
            jax.ShapeDtypeStruct((4, _ROWS, _LANES), f32),
            jax.ShapeDtypeStruct((_ROWS, _LANES), f32),
        ],
    )(x, prop, w6, b6, w7, b7, wc, bc, wr, br, bounds)


def _nms_body(bxT_ref, msT_ref, outb_ref, outs_ref, outl_ref,
              nbT_ref, sT_ref, a2T_ref, rmax_ref, rix_ref):
    big = jnp.int32(2**30)
    m = jnp.max(bxT_ref[...])
    c_col = jax.lax.broadcasted_iota(jnp.int32, (_LANES, 1), 0)
    offsT = (c_col + 1).astype(jnp.float32) * (m + 1.0)
    for k in range(4):
        nbT_ref[k] = bxT_ref[k] + offsT
    a2T_ref[...] = (nbT_ref[2] - nbT_ref[0]) * (nbT_ref[3] - nbT_ref[1])
    sT_ref[...] = msT_ref[...]
    piota = jax.lax.broadcasted_iota(jnp.int32, (_LANES, _ROWS), 1)
    rmax_ref[...] = jnp.max(sT_ref[...], axis=1, keepdims=True)
    rix_ref[...] = jnp.min(
        jnp.where(sT_ref[...] == rmax_ref[...], piota, big), axis=1,
        keepdims=True)
    prow = jax.lax.broadcasted_iota(jnp.int32, (1, _ROWS), 1)

    def body(i, carry):
        rmax = rmax_ref[...]
        mv = jnp.max(rmax)
        cand = jnp.where(rmax == mv, rix_ref[...] * 90 + c_col, big)
        idx = jnp.min(cand)
        ok = mv > 0.0
        n = idx // 90
        c = idx - n * 90
        sel = prow == n

        def ext(row):
            return jnp.sum(jnp.where(sel, row, 0.0))

        nrow = [nbT_ref[k, pl.ds(c, 1), :] for k in range(4)]
        px = [ext(r) for r in nrow]
        rx = [ext(bxT_ref[k, pl.ds(c, 1), :]) for k in range(4)]
        area1 = (px[2] - px[0]) * (px[3] - px[1])
        ltx = jnp.maximum(px[0], nrow[0])
        lty = jnp.maximum(px[1], nrow[1])
        rbx = jnp.minimum(px[2], nrow[2])
        rby = jnp.minimum(px[3], nrow[3])
        w = jnp.clip(rbx - ltx, 0.0, None)
        h = jnp.clip(rby - lty, 0.0, None)
        inter = w * h
        iou = inter / (area1 + a2T_ref[pl.ds(c, 1), :] - inter + 1e-9)
        kill = (ok & (iou > _NMS_THRESH)) | sel
        newrow = jnp.where(kill, -1.0, sT_ref[pl.ds(c, 1), :])
        sT_ref[pl.ds(c, 1), :] = newrow
        newmax = jnp.max(newrow)
        newrix = jnp.min(jnp.where(newrow == newmax, prow, big))
        rmax_ref[pl.ds(c, 1), :] = jnp.full((1, 1), 0.0) + newmax
        rix_ref[pl.ds(c, 1), :] = jnp.full((1, 1), 0) + newrix
        for k in range(4):
            outb_ref[k, pl.ds(i, 1), :] = jnp.broadcast_to(
                jnp.where(ok, rx[k], 0.0), (1, _LANES))
        outs_ref[pl.ds(i, 1), :] = jnp.broadcast_to(
            jnp.where(ok, mv, 0.0), (1, _LANES))
        lab = jnp.where(ok, c + 1, 0)
        outl_ref[pl.ds(i, 1), :] = jnp.broadcast_to(lab, (1, _LANES))
        return carry

    jax.lax.fori_loop(0, _DETS, body, 0)


@jax.jit
def _nms_pallas(bx, ms):
    f32 = jnp.float32
    bxT = jnp.transpose(bx, (0, 2, 1))
    msT = ms.T
    return pl.pallas_call(
        _nms_body,
        out_shape=[
            jax.ShapeDtypeStruct((4, _DETS, _LANES), f32),
            jax.ShapeDtypeStruct((_DETS, _LANES), f32),
            jax.ShapeDtypeStruct((_DETS, _LANES), jnp.int32),
        ],
        scratch_shapes=[
            pltpu.VMEM((4, _LANES, _ROWS), f32),
            pltpu.VMEM((_LANES, _ROWS), f32),
            pltpu.VMEM((_LANES, _ROWS), f32),
            pltpu.VMEM((_LANES, 1), f32),
            pltpu.VMEM((_LANES, 1), jnp.int32),
        ],
    )(bxT, msT)


def kernel(features, proposals, fc6_w, fc6_b, fc7_w, fc7_b, cls_w, cls_b,
           reg_w, reg_b, img_h, img_w):
    f32 = jnp.float32
    feat = features[0]
    prop = jnp.concatenate(
        [proposals, jnp.zeros((_NPROPP - _NPROP, 4), f32)], axis=0)
    table = feat.transpose(1, 2, 0).reshape(2500, 64)
    pooled = _sc_roi(table, prop)
    x = pooled.reshape(_NPROPP, 3136)
    b6 = fc6_b.reshape(1, 256)
    b7 = fc7_b.reshape(1, 256)
    wc = jnp.concatenate([cls_w, jnp.zeros((256, _LANES - _NCLS), f32)], axis=1)
    bc = jnp.concatenate([cls_b, jnp.zeros((_LANES - _NCLS,), f32)]).reshape(1, _LANES)
    wr4 = reg_w.reshape(256, _NCLS, 4).transpose(0, 2, 1)[:, :, 1:]
    wr = jnp.concatenate([wr4, jnp.zeros((256, 4, 6), f32)], axis=2).reshape(256, 384)
    br4 = reg_b.reshape(_NCLS, 4).T[:, 1:]
    br = jnp.concatenate([br4, jnp.zeros((4, 6), f32)], axis=1).reshape(1, 384)
    bounds = jnp.zeros((1, _LANES), f32)
    bounds = bounds.at[0, 0].set(jnp.asarray(img_w).astype(f32))
    bounds = bounds.at[0, 1].set(jnp.asarray(img_h).astype(f32))
    w6p = fc6_w.reshape(64, 49, 256).transpose(1, 0, 2).reshape(3136, 256)
    bx, ms = _head_pallas(x, prop, w6p, b6, fc7_w, b7, wc, bc, wr, br, bounds)
    outb, outs, outl = _nms_pallas(bx, ms)
    out_boxes = outb[:, :, 0].T
    out_scores = outs[:, 0]
    out_labels = outl[:, 0]
    return out_boxes, out_scores, out_labels
